# baseline (speedup 1.0000x reference)
import functools

import jax
import jax.numpy as jnp
import numpy as np
from jax import lax
from jax.experimental import pallas as pl
from jax.experimental.pallas import tpu as pltpu
from jax.experimental.pallas import tpu_sc as plsc

_LABEL_DIM = 18
_NUM_NEGS = 10
_N_POSS = 120
_LOG_HALF = float(np.log(0.5))


@functools.lru_cache(maxsize=None)
def _sample_tables(n_rows: int):
    rng = np.random.default_rng(0)
    g = rng.gumbel(size=(n_rows, _N_POSS))
    s0 = np.argsort(-g, axis=1)[:, :_NUM_NEGS]
    repls = [rng.integers(0, _N_POSS, size=(n_rows, _NUM_NEGS))
             for _ in range(16)]
    alt = repls[15].copy()
    decided = np.zeros((n_rows, _NUM_NEGS), dtype=bool)
    for m in range(15):
        take = (~decided) & (repls[m] != s0)
        alt[take] = repls[m][take]
        decided |= take
    c0 = np.zeros((n_rows, _LABEL_DIM), dtype=np.float32)
    a, b, c = s0 // 60, (s0 // 10) % 6, s0 % 10
    for j in range(_NUM_NEGS):
        np.add.at(c0, (np.arange(n_rows), a[:, j]), 1.0)
        np.add.at(c0, (np.arange(n_rows), 2 + b[:, j]), 1.0)
        np.add.at(c0, (np.arange(n_rows), 8 + c[:, j]), 1.0)
    packed = (s0 + 128 * alt).astype(np.int32)
    return (packed, c0)


def _decode_np():
    w = np.zeros((_LABEL_DIM,), dtype=np.float32)
    w[0:2] = 60.0 * np.arange(2)
    w[2:8] = 10.0 * np.arange(6)
    w[8:18] = 1.0 * np.arange(10)
    return w


def _sc_sample(y, packed):
    bsz = y.shape[0]
    info = plsc.get_sparse_core_info()
    nw = info.num_cores * info.num_subcores
    rows = bsz // nw
    dec = _decode_np()
    mesh = plsc.VectorSubcoreMesh(core_axis_name="c", subcore_axis_name="s")

    @functools.partial(
        pl.kernel, mesh=mesh,
        compiler_params=pltpu.CompilerParams(needs_layout_passes=False),
        out_type=jax.ShapeDtypeStruct((bsz * 2,), jnp.float32),
        scratch_types=[
            pltpu.VMEM((rows * _LABEL_DIM,), jnp.float32),
            pltpu.VMEM((rows * _NUM_NEGS,), jnp.int32),
            pltpu.VMEM((rows * 2,), jnp.float32),
        ],
    )
    def sample(y_hbm, pk_hbm, out_hbm, y_v, pk_v, o_v):
        wid = lax.axis_index("s") * info.num_cores + lax.axis_index("c")
        base = wid * rows
        pltpu.sync_copy(y_hbm.at[pl.ds(base * _LABEL_DIM,
                                       rows * _LABEL_DIM)], y_v)
        pltpu.sync_copy(pk_hbm.at[pl.ds(base * _NUM_NEGS,
                                        rows * _NUM_NEGS)], pk_v)

        def group(g2, carry):
            row = g2 * 16 + lax.broadcasted_iota(jnp.int32, (16,), 0)
            row18 = row * _LABEL_DIM
            row10 = row * _NUM_NEGS
            tgt = jnp.zeros((16,), jnp.float32)
            for k in range(_LABEL_DIM):
                if dec[k] != 0.0:
                    v = plsc.load_gather(y_v, [row18 + k])
                    tgt = tgt + v * float(dec[k])
            ti = tgt.astype(jnp.int32)
            hc = jnp.zeros((16,), jnp.float32)
            av = jnp.zeros((16,), jnp.float32)
            for j in range(_NUM_NEGS):
                pk = plsc.load_gather(pk_v, [row10 + j])
                m = ti == pk % 128
                hc = hc + jnp.where(m, 1.0, 0.0)
                av = av + jnp.where(m, (pk // 128).astype(jnp.float32), 0.0)
            plsc.store_scatter(o_v, [row * 2], hc)
            plsc.store_scatter(o_v, [row * 2 + 1], av)
            return carry

        lax.fori_loop(0, rows // 16, group, 0)
        pltpu.sync_copy(o_v, out_hbm.at[pl.ds(base * 2, rows * 2)])

    out = sample(y.reshape(bsz * _LABEL_DIM), packed.reshape(-1))
    return out.reshape(bsz, 2)


def _tc_body(x_ref, y_ref, ob_ref, w1_ref, b1_ref, w2_ref, b2_ref, w3_ref,
             b3_ref, hcav_ref, c0_ref, wu_ref, loss_ref,
             c_all_ref, acc_ref, r_ref):
    g = pl.program_id(0)
    nsteps = pl.num_programs(0)
    blk = x_ref.shape[0]

    @pl.when(g == 0)
    def _init():
        acc_ref[0] = 0.0
        acc_ref[1] = 0.0
        acc_ref[2] = 0.0
        r_ref[0] = 0

    xb = x_ref[...]
    h = jax.nn.sigmoid(jnp.dot(xb, w1_ref[...]) + b1_ref[...])
    h = jax.nn.sigmoid(jnp.dot(h, w2_ref[...]) + b2_ref[...])
    wu = jnp.dot(h, w3_ref[...]) + b3_ref[...]
    wu_ref[...] = wu

    yb = y_ref[...]
    wc = wu * ob_ref[...]
    rowsum = jnp.sum(wc, axis=1)
    maskf = (rowsum != 0.0).astype(jnp.float32)
    pos = jnp.sum(jax.nn.log_sigmoid(wc * yb), axis=1)
    lmat = jax.nn.log_sigmoid(-wc) - _LOG_HALF

    hcf = hcav_ref[:, 0:1]
    av = hcav_ref[:, 1:2].astype(jnp.int32)
    kio = jax.lax.broadcasted_iota(jnp.int32, (blk, _LABEL_DIM), 1)
    hot_av = ((kio == av // 60).astype(jnp.float32)
              + (kio == (av // 10) % 6 + 2).astype(jnp.float32)
              + (kio == av % 10 + 8).astype(jnp.float32))
    cmat = c0_ref[...] + hcf * (hot_av - yb)
    cpad = jnp.concatenate(
        [cmat, jnp.zeros((8, _LABEL_DIM), jnp.float32)], axis=0)
    c_all_ref[pl.ds(pl.multiple_of(g * blk, blk), blk + 8), :] = cpad

    nvb = jnp.sum(maskf)
    r0 = r_ref[0]

    def _fast(_):
        return jnp.sum(cmat * lmat)

    def _slow(_):
        ra = (r0 // 8) * 8
        d = r0 - ra
        io_r = jax.lax.broadcasted_iota(jnp.int32, (blk + 8, blk), 0)
        io_c = jax.lax.broadcasted_iota(jnp.int32, (blk, blk), 1)
        tri = (jax.lax.broadcasted_iota(jnp.int32, (blk, blk), 0)
               > io_c).astype(jnp.float32)
        lrank = jnp.dot(tri, maskf[:, None])[:, 0]
        lrank_i = lrank.astype(jnp.int32) + d
        perm = ((io_r == lrank_i[None, :]).astype(jnp.float32)
                * maskf[None, :])
        lc = jnp.dot(perm, lmat)
        window = c_all_ref[pl.ds(pl.multiple_of(ra, 8), blk + 8), :]
        return jnp.sum(window * lc)

    all_valid = jnp.logical_and(r0 == g * blk,
                                nvb.astype(jnp.int32) == blk)
    hard = jax.lax.cond(all_valid, _fast, _slow, 0)
    acc_ref[0] += jnp.sum(pos * maskf)
    acc_ref[1] += hard
    acc_ref[2] += nvb
    r_ref[0] = r0 + nvb.astype(jnp.int32)

    @pl.when(g == nsteps - 1)
    def _fin():
        nv = acc_ref[2]
        neg_const = nv * (_NUM_NEGS * _LABEL_DIM * _LOG_HALF)
        loss_ref[0, 0] = -(acc_ref[0] + acc_ref[1] + neg_const) / nv


def kernel(x, y, ob, W1, b1, W2, b2, W3, b3, all_possible):
    del all_possible
    bsz, d_user = x.shape
    blk = 2048
    nsteps = bsz // blk
    packed_np, c0_np = _sample_tables(bsz)
    packed = jnp.asarray(packed_np)
    c0 = jnp.asarray(c0_np)

    hcav = _sc_sample(y, packed)

    wu, loss = pl.pallas_call(
        _tc_body,
        grid=(nsteps,),
        in_specs=[
            pl.BlockSpec((blk, d_user), lambda g: (g, 0)),
            pl.BlockSpec((blk, _LABEL_DIM), lambda g: (g, 0)),
            pl.BlockSpec((blk, _LABEL_DIM), lambda g: (g, 0)),
            pl.BlockSpec(W1.shape, lambda g: (0, 0)),
            pl.BlockSpec((1, b1.shape[0]), lambda g: (0, 0)),
            pl.BlockSpec(W2.shape, lambda g: (0, 0)),
            pl.BlockSpec((1, b2.shape[0]), lambda g: (0, 0)),
            pl.BlockSpec(W3.shape, lambda g: (0, 0)),
            pl.BlockSpec((1, b3.shape[0]), lambda g: (0, 0)),
            pl.BlockSpec((blk, 2), lambda g: (g, 0)),
            pl.BlockSpec((blk, _LABEL_DIM), lambda g: (g, 0)),
        ],
        out_specs=[
            pl.BlockSpec((blk, _LABEL_DIM), lambda g: (g, 0)),
            pl.BlockSpec((1, 1), lambda g: (0, 0), memory_space=pltpu.SMEM),
        ],
        out_shape=[
            jax.ShapeDtypeStruct((bsz, _LABEL_DIM), jnp.float32),
            jax.ShapeDtypeStruct((1, 1), jnp.float32),
        ],
        scratch_shapes=[
            pltpu.VMEM((bsz + 8, _LABEL_DIM), jnp.float32),
            pltpu.SMEM((3,), jnp.float32),
            pltpu.SMEM((1,), jnp.int32),
        ],
    )(x, y, ob, W1, b1.reshape(1, -1), W2, b2.reshape(1, -1), W3,
      b3.reshape(1, -1), hcav, c0)
    return (wu, loss[0, 0])


# speedup vs baseline: 1.0735x; 1.0735x over previous
import functools

import jax
import jax.numpy as jnp
import numpy as np
from jax import lax
from jax.experimental import pallas as pl
from jax.experimental.pallas import tpu as pltpu
from jax.experimental.pallas import tpu_sc as plsc

_LABEL_DIM = 18
_NUM_NEGS = 10
_N_POSS = 120
_LOG_HALF = float(np.log(0.5))


@functools.lru_cache(maxsize=None)
def _sample_tables(n_rows: int):
    rng = np.random.default_rng(0)
    g = rng.gumbel(size=(n_rows, _N_POSS))
    s0 = np.argsort(-g, axis=1)[:, :_NUM_NEGS]
    repls = [rng.integers(0, _N_POSS, size=(n_rows, _NUM_NEGS))
             for _ in range(16)]
    alt = repls[15].copy()
    decided = np.zeros((n_rows, _NUM_NEGS), dtype=bool)
    for m in range(15):
        take = (~decided) & (repls[m] != s0)
        alt[take] = repls[m][take]
        decided |= take
    c0 = np.zeros((n_rows, _LABEL_DIM), dtype=np.float32)
    a, b, c = s0 // 60, (s0 // 10) % 6, s0 % 10
    for j in range(_NUM_NEGS):
        np.add.at(c0, (np.arange(n_rows), a[:, j]), 1.0)
        np.add.at(c0, (np.arange(n_rows), 2 + b[:, j]), 1.0)
        np.add.at(c0, (np.arange(n_rows), 8 + c[:, j]), 1.0)
    packed = (s0 + 128 * alt).astype(np.int32)
    return (packed, c0)


def _decode_np():
    w = np.zeros((_LABEL_DIM,), dtype=np.float32)
    w[0:2] = 60.0 * np.arange(2)
    w[2:8] = 10.0 * np.arange(6)
    w[8:18] = 1.0 * np.arange(10)
    return w


def _sc_counts(y, packed, c0):
    bsz = y.shape[0]
    info = plsc.get_sparse_core_info()
    nw = info.num_cores * info.num_subcores
    rows = bsz // nw
    dec = _decode_np()
    mesh = plsc.VectorSubcoreMesh(core_axis_name="c", subcore_axis_name="s")

    @functools.partial(
        pl.kernel, mesh=mesh,
        compiler_params=pltpu.CompilerParams(needs_layout_passes=False),
        out_type=jax.ShapeDtypeStruct((bsz * _LABEL_DIM,), jnp.float32),
        scratch_types=[
            pltpu.VMEM((rows * _LABEL_DIM,), jnp.float32),
            pltpu.VMEM((rows * _NUM_NEGS,), jnp.int32),
            pltpu.VMEM((rows * _LABEL_DIM,), jnp.float32),
            pltpu.VMEM((rows * _LABEL_DIM,), jnp.float32),
        ],
    )
    def counts(y_hbm, pk_hbm, c0_hbm, out_hbm, y_v, pk_v, c0_v, o_v):
        wid = lax.axis_index("s") * info.num_cores + lax.axis_index("c")
        base = wid * rows
        pltpu.sync_copy(y_hbm.at[pl.ds(base * _LABEL_DIM,
                                       rows * _LABEL_DIM)], y_v)
        pltpu.sync_copy(pk_hbm.at[pl.ds(base * _NUM_NEGS,
                                        rows * _NUM_NEGS)], pk_v)
        pltpu.sync_copy(c0_hbm.at[pl.ds(base * _LABEL_DIM,
                                        rows * _LABEL_DIM)], c0_v)

        def group(g2, carry):
            row = g2 * 16 + lax.broadcasted_iota(jnp.int32, (16,), 0)
            row18 = row * _LABEL_DIM
            row10 = row * _NUM_NEGS
            tgt = jnp.zeros((16,), jnp.float32)
            yk = []
            for k in range(_LABEL_DIM):
                v = plsc.load_gather(y_v, [row18 + k])
                yk.append(v)
                if dec[k] != 0.0:
                    tgt = tgt + v * float(dec[k])
            ti = tgt.astype(jnp.int32)
            hc = jnp.zeros((16,), jnp.float32)
            av = jnp.zeros((16,), jnp.float32)
            for j in range(_NUM_NEGS):
                pk = plsc.load_gather(pk_v, [row10 + j])
                m = ti == pk % 128
                hc = hc + jnp.where(m, 1.0, 0.0)
                av = av + jnp.where(m, (pk // 128).astype(jnp.float32), 0.0)
            avi = av.astype(jnp.int32)
            ai = avi // 60
            bi = (avi // 10) % 6
            ci = avi % 10
            for k in range(_LABEL_DIM):
                if k < 2:
                    hot = jnp.where(ai == k, 1.0, 0.0)
                elif k < 8:
                    hot = jnp.where(bi == k - 2, 1.0, 0.0)
                else:
                    hot = jnp.where(ci == k - 8, 1.0, 0.0)
                c0e = plsc.load_gather(c0_v, [row18 + k])
                plsc.store_scatter(o_v, [row18 + k],
                                   c0e + hc * (hot - yk[k]))
            return carry

        lax.fori_loop(0, rows // 16, group, 0)
        pltpu.sync_copy(o_v, out_hbm.at[pl.ds(base * _LABEL_DIM,
                                              rows * _LABEL_DIM)])

    out = counts(y.reshape(bsz * _LABEL_DIM), packed.reshape(-1),
                 c0.reshape(-1))
    return out.reshape(bsz, _LABEL_DIM)


def _tc_body(x_ref, y_ref, ob_ref, w1_ref, b1_ref, w2_ref, b2_ref, w3_ref,
             b3_ref, c_ref, wu_ref, loss_ref,
             c_all_ref, acc_ref, r_ref):
    g = pl.program_id(0)
    nsteps = pl.num_programs(0)
    blk = x_ref.shape[0]

    @pl.when(g == 0)
    def _init():
        acc_ref[0] = 0.0
        acc_ref[1] = 0.0
        acc_ref[2] = 0.0
        r_ref[0] = 0

    xb = x_ref[...]
    h = jax.nn.sigmoid(jnp.dot(xb, w1_ref[...]) + b1_ref[...])
    h = jax.nn.sigmoid(jnp.dot(h, w2_ref[...]) + b2_ref[...])
    wu = jnp.dot(h, w3_ref[...]) + b3_ref[...]
    wu_ref[...] = wu

    yb = y_ref[...]
    wc = wu * ob_ref[...]
    rowsum = jnp.sum(wc, axis=1)
    maskf = (rowsum != 0.0).astype(jnp.float32)
    pos = jnp.sum(jax.nn.log_sigmoid(wc * yb), axis=1)
    lmat = jax.nn.log_sigmoid(-wc) - _LOG_HALF

    cmat = c_ref[...]
    cpad = jnp.concatenate(
        [cmat, jnp.zeros((8, _LABEL_DIM), jnp.float32)], axis=0)
    c_all_ref[pl.ds(pl.multiple_of(g * blk, blk), blk + 8), :] = cpad

    nvb = jnp.sum(maskf)
    r0 = r_ref[0]

    def _fast(_):
        return jnp.sum(cmat * lmat)

    def _slow(_):
        ra = (r0 // 8) * 8
        d = r0 - ra
        io_r = jax.lax.broadcasted_iota(jnp.int32, (blk + 8, blk), 0)
        io_c = jax.lax.broadcasted_iota(jnp.int32, (blk, blk), 1)
        tri = (jax.lax.broadcasted_iota(jnp.int32, (blk, blk), 0)
               > io_c).astype(jnp.float32)
        lrank = jnp.dot(tri, maskf[:, None])[:, 0]
        lrank_i = lrank.astype(jnp.int32) + d
        perm = ((io_r == lrank_i[None, :]).astype(jnp.float32)
                * maskf[None, :])
        lc = jnp.dot(perm, lmat)
        window = c_all_ref[pl.ds(pl.multiple_of(ra, 8), blk + 8), :]
        return jnp.sum(window * lc)

    all_valid = jnp.logical_and(r0 == g * blk,
                                nvb.astype(jnp.int32) == blk)
    hard = jax.lax.cond(all_valid, _fast, _slow, 0)
    acc_ref[0] += jnp.sum(pos * maskf)
    acc_ref[1] += hard
    acc_ref[2] += nvb
    r_ref[0] = r0 + nvb.astype(jnp.int32)

    @pl.when(g == nsteps - 1)
    def _fin():
        nv = acc_ref[2]
        neg_const = nv * (_NUM_NEGS * _LABEL_DIM * _LOG_HALF)
        loss_ref[0, 0] = -(acc_ref[0] + acc_ref[1] + neg_const) / nv


def kernel(x, y, ob, W1, b1, W2, b2, W3, b3, all_possible):
    del all_possible
    bsz, d_user = x.shape
    blk = 2048
    nsteps = bsz // blk
    packed_np, c0_np = _sample_tables(bsz)
    packed = jnp.asarray(packed_np)
    c0 = jnp.asarray(c0_np)

    cmat = _sc_counts(y, packed, c0)

    wu, loss = pl.pallas_call(
        _tc_body,
        grid=(nsteps,),
        in_specs=[
            pl.BlockSpec((blk, d_user), lambda g: (g, 0)),
            pl.BlockSpec((blk, _LABEL_DIM), lambda g: (g, 0)),
            pl.BlockSpec((blk, _LABEL_DIM), lambda g: (g, 0)),
            pl.BlockSpec(W1.shape, lambda g: (0, 0)),
            pl.BlockSpec((1, b1.shape[0]), lambda g: (0, 0)),
            pl.BlockSpec(W2.shape, lambda g: (0, 0)),
            pl.BlockSpec((1, b2.shape[0]), lambda g: (0, 0)),
            pl.BlockSpec(W3.shape, lambda g: (0, 0)),
            pl.BlockSpec((1, b3.shape[0]), lambda g: (0, 0)),
            pl.BlockSpec((blk, _LABEL_DIM), lambda g: (g, 0)),
        ],
        out_specs=[
            pl.BlockSpec((blk, _LABEL_DIM), lambda g: (g, 0)),
            pl.BlockSpec((1, 1), lambda g: (0, 0), memory_space=pltpu.SMEM),
        ],
        out_shape=[
            jax.ShapeDtypeStruct((bsz, _LABEL_DIM), jnp.float32),
            jax.ShapeDtypeStruct((1, 1), jnp.float32),
        ],
        scratch_shapes=[
            pltpu.VMEM((bsz + 8, _LABEL_DIM), jnp.float32),
            pltpu.SMEM((3,), jnp.float32),
            pltpu.SMEM((1,), jnp.int32),
        ],
    )(x, y, ob, W1, b1.reshape(1, -1), W2, b2.reshape(1, -1), W3,
      b3.reshape(1, -1), cmat)
    return (wu, loss[0, 0])
